# baseline (speedup 1.0000x reference)
import jax
import jax.numpy as jnp
from jax.experimental import pallas as pl
from jax.experimental.pallas import tpu as pltpu

_NEG = -1e30


def _gelu(x):
    return 0.5 * x * (1.0 + jax.lax.erf(x * 0.7071067811865476))


def _gate_body(E, x_ref, p_ref, t_ref, tw1_ref, tb1_ref, tw2_ref, tb2_ref,
               gw_ref, aux_ref, stats_ref, loss_ref):
    step = pl.program_id(0)
    nsteps = pl.num_programs(0)
    t_col = t_ref[:, 0:1]
    th = t_col * tw1_ref[...] + tb1_ref[...]
    temb = jnp.dot(_gelu(th), tw2_ref[...],
                   preferred_element_type=jnp.float32) + tb2_ref[...]
    logits = (jnp.dot(x_ref[...], gw_ref[0], preferred_element_type=jnp.float32)
              + jnp.dot(p_ref[...], gw_ref[1], preferred_element_type=jnp.float32)
              + jnp.dot(temb, gw_ref[2], preferred_element_type=jnp.float32))
    lane = jax.lax.broadcasted_iota(jnp.int32, logits.shape, 1)
    lm = jnp.where(lane < E, logits, _NEG)
    m0 = jnp.max(lm, axis=1, keepdims=True)
    i0 = jnp.min(jnp.where(lm == m0, lane, 1 << 20), axis=1, keepdims=True)
    lm2 = jnp.where(lane == i0, _NEG, lm)
    m1 = jnp.max(lm2, axis=1, keepdims=True)
    i1 = jnp.min(jnp.where(lm2 == m1, lane, 1 << 20), axis=1, keepdims=True)
    b = jnp.exp(m1 - m0)
    denom = 1.0 + b
    s0 = 1.0 / denom
    s1 = b / denom
    swap = i1 < i0
    e_lo = jnp.where(swap, i1, i0)
    e_hi = jnp.where(swap, i0, i1)
    g_lo = jnp.where(swap, s1, s0)
    g_hi = jnp.where(swap, s0, s1)
    aux = (jnp.where(lane == 0, t_col, 0.0)
           + jnp.where(lane == 1, g_lo, 0.0)
           + jnp.where(lane == 2, g_hi, 0.0))
    aux_ref[...] = aux
    hot_lo = lane == e_lo
    hot_hi = lane == e_hi
    cnt = (jnp.sum(jnp.where(hot_lo & (g_lo > 0.0), 1.0, 0.0), axis=0, keepdims=True)
           + jnp.sum(jnp.where(hot_hi & (g_hi > 0.0), 1.0, 0.0), axis=0, keepdims=True))
    ld = (jnp.sum(jnp.where(hot_lo, g_lo, 0.0), axis=0, keepdims=True)
          + jnp.sum(jnp.where(hot_hi, g_hi, 0.0), axis=0, keepdims=True))
    upd = jnp.concatenate([cnt, ld], axis=0)

    @pl.when(step == 0)
    def _():
        stats_ref[...] = jnp.zeros_like(stats_ref)

    stats_ref[...] += upd

    @pl.when(step == nsteps - 1)
    def _():
        ldv = stats_ref[1:2, :]
        lane2 = jax.lax.broadcasted_iota(jnp.int32, ldv.shape, 1)
        msk = lane2 < E
        tot = jnp.sum(jnp.where(msk, ldv, 0.0))
        mean = tot / E
        var = jnp.sum(jnp.where(msk, (ldv - mean) ** 2, 0.0)) / (E - 1)
        loss_ref[...] = jnp.full((1, 1), 2.0 * var / (mean * mean + 1e-10),
                                 jnp.float32)


def _expert_body(D, S, HC, last_half,
                 jbv, ebv, firstv, activev, newwv, cume,
                 x_ref, aux_ref, tw2_ref, ep_ref, w1_ref, w2_ref,
                 out_ref, w1s_ref, w2s_ref, tws_ref):
    g = pl.program_id(0)

    @pl.when(newwv[g] == 1)
    def _():
        w1s_ref[...] = w1_ref[0].astype(jnp.bfloat16)
        w2s_ref[...] = w2_ref[0].astype(jnp.bfloat16)
        tws_ref[...] = tw2_ref[0].astype(jnp.bfloat16)

    @pl.when(firstv[g] == 1)
    def _():
        out_ref[...] = jnp.zeros_like(out_ref)

    @pl.when(activev[g] == 1)
    def _():
        tw1 = ep_ref[0, 0:1, :]
        tb1 = ep_ref[0, 1:2, :]
        tb2 = ep_ref[0, 2:3, :]
        b2 = ep_ref[0, 3:4, :]
        b1 = ep_ref[0, 4:, :].reshape(1, 4 * D)[:, (HC if last_half else 0):][:, :HC]
        t_col = aux_ref[:, 0:1]
        th = t_col * tw1 + tb1
        temb = jnp.dot(_gelu(th).astype(jnp.bfloat16), tws_ref[...],
                       preferred_element_type=jnp.float32) + tb2
        e = ebv[g]
        lo = cume[e]
        hi = cume[e + 1]
        j = jbv[g]
        r0 = 2 * j * S + 2 * jax.lax.broadcasted_iota(jnp.int32, (S, 1), 0)
        r1 = r0 + 1
        gl = aux_ref[:, 1:2]
        gh = aux_ref[:, 2:3]
        w = (jnp.where((r0 >= lo) & (r0 < hi), gl, 0.0)
             + jnp.where((r1 >= lo) & (r1 < hi), gh, 0.0))
        xb = x_ref[...].astype(jnp.bfloat16)
        hh = _gelu(jnp.dot(xb, w1s_ref[:D, :], preferred_element_type=jnp.float32)
                   + jnp.dot(temb.astype(jnp.bfloat16), w1s_ref[D:, :],
                             preferred_element_type=jnp.float32)
                   + b1)
        oe = jnp.dot(hh.astype(jnp.bfloat16), w2s_ref[...],
                     preferred_element_type=jnp.float32)
        if last_half:
            oe = oe + b2
        out_ref[...] += w * oe


def kernel(x, prompt, t, te_w1, te_b1, te_w2, te_b2, gate_w,
           ex_te_w1, ex_te_b1, ex_te_w2, ex_te_b2,
           ex_m_w1, ex_m_b1, ex_m_w2, ex_m_b2):
    Bq, Nq, Dq = x.shape
    T = Bq * Nq
    D = Dq
    E = gate_w.shape[1]
    H = ex_m_w1.shape[2]

    SA = 256
    S = 512
    J = T // S
    G = J + E - 1

    xf = x.reshape(T, D)
    pf = prompt.reshape(T, D)
    t128 = jnp.broadcast_to(t.reshape(T, 1), (T, 128))
    gw3 = jnp.zeros((3, D, 128), jnp.float32).at[:, :, :E].set(
        gate_w.reshape(3, D, E))

    aux, stats, loss = pl.pallas_call(
        lambda *a: _gate_body(E, *a),
        grid=(T // SA,),
        in_specs=[
            pl.BlockSpec((SA, D), lambda i: (i, 0)),
            pl.BlockSpec((SA, D), lambda i: (i, 0)),
            pl.BlockSpec((SA, 128), lambda i: (i, 0)),
            pl.BlockSpec((1, D), lambda i: (0, 0)),
            pl.BlockSpec((1, D), lambda i: (0, 0)),
            pl.BlockSpec((D, D), lambda i: (0, 0)),
            pl.BlockSpec((1, D), lambda i: (0, 0)),
            pl.BlockSpec((3, D, 128), lambda i: (0, 0, 0)),
        ],
        out_specs=[
            pl.BlockSpec((SA, 128), lambda i: (i, 0)),
            pl.BlockSpec((2, 128), lambda i: (0, 0)),
            pl.BlockSpec((1, 1), lambda i: (0, 0)),
        ],
        out_shape=[
            jax.ShapeDtypeStruct((T, 128), jnp.float32),
            jax.ShapeDtypeStruct((2, 128), jnp.float32),
            jax.ShapeDtypeStruct((1, 1), jnp.float32),
        ],
    )(xf, pf, t128, te_w1, te_b1.reshape(1, D), te_w2, te_b2.reshape(1, D), gw3)

    counts = stats[0, :E].astype(jnp.int32)
    cum = jnp.cumsum(counts, dtype=jnp.int32)
    cume = jnp.concatenate([jnp.zeros((1,), jnp.int32), cum])
    rows_start = (2 * S) * jnp.arange(J, dtype=jnp.int32)
    ef = jnp.searchsorted(cum, rows_start, side='right').astype(jnp.int32)
    el = jnp.searchsorted(cum, rows_start + 2 * S - 1, side='right').astype(jnp.int32)
    ef_c = jnp.minimum(ef, E - 1)
    el_c = jnp.minimum(el, E - 1)
    npj = el_c - ef_c + 1
    off = jnp.concatenate(
        [jnp.zeros((1,), jnp.int32), jnp.cumsum(npj, dtype=jnp.int32)])
    total = off[-1]
    garr = jnp.arange(G, dtype=jnp.int32)
    jb = jnp.clip(jnp.searchsorted(off, garr, side='right').astype(jnp.int32) - 1,
                  0, J - 1)
    pin = garr - off[jb]
    eb = jnp.minimum(ef_c[jb] + pin, el_c[jb])
    active = ((garr < total) & (ef[jb] < E)).astype(jnp.int32)
    first = ((garr == off[jb]) & (garr < total)).astype(jnp.int32)

    neww = jnp.concatenate(
        [jnp.ones((1,), jnp.int32), (eb[1:] != eb[:-1]).astype(jnp.int32)])

    HC = H // 2
    epack = jnp.concatenate([
        ex_te_w1.reshape(E, 1, D),
        ex_te_b1.reshape(E, 1, D),
        ex_te_b2.reshape(E, 1, D),
        ex_m_b2.reshape(E, 1, D),
        ex_m_b1.reshape(E, 4, D),
    ], axis=1)

    def half_call(c):
        grid_spec = pltpu.PrefetchScalarGridSpec(
            num_scalar_prefetch=6,
            grid=(G,),
            in_specs=[
                pl.BlockSpec((S, D), lambda g, jv, ev, fv, av, nv, cm: (jv[g], 0)),
                pl.BlockSpec((S, 128), lambda g, jv, ev, fv, av, nv, cm: (jv[g], 0)),
                pl.BlockSpec((1, D, D), lambda g, jv, ev, fv, av, nv, cm: (ev[g], 0, 0)),
                pl.BlockSpec((1, 8, D), lambda g, jv, ev, fv, av, nv, cm: (ev[g], 0, 0)),
                pl.BlockSpec((1, 2 * D, HC),
                             lambda g, jv, ev, fv, av, nv, cm: (ev[g], 0, c)),
                pl.BlockSpec((1, HC, D),
                             lambda g, jv, ev, fv, av, nv, cm: (ev[g], c, 0)),
            ],
            out_specs=pl.BlockSpec((S, D), lambda g, jv, ev, fv, av, nv, cm: (jv[g], 0)),
            scratch_shapes=[
                pltpu.VMEM((2 * D, HC), jnp.bfloat16),
                pltpu.VMEM((HC, D), jnp.bfloat16),
                pltpu.VMEM((D, D), jnp.bfloat16),
            ],
        )
        return pl.pallas_call(
            lambda *a: _expert_body(D, S, HC, c == 1, *a),
            grid_spec=grid_spec,
            out_shape=jax.ShapeDtypeStruct((T, D), jnp.float32),
        )(jb, eb, first, active, neww, cume,
          xf, aux, ex_te_w2, epack, ex_m_w1, ex_m_w2)

    out = half_call(0) + half_call(1)

    output = out.reshape(Bq, Nq, Dq)
    moe_loss = loss[0, 0]
    return (output, moe_loss)


# speedup vs baseline: 1.0312x; 1.0312x over previous
import jax
import jax.numpy as jnp
from jax.experimental import pallas as pl
from jax.experimental.pallas import tpu as pltpu

_NEG = -1e30


def _gelu(x):
    return 0.5 * x * (1.0 + jax.lax.erf(x * 0.7071067811865476))


def _gate_body(E, x_ref, p_ref, t_ref, tw1_ref, tb1_ref, tw2_ref, tb2_ref,
               gw_ref, aux_ref, stats_ref, loss_ref):
    step = pl.program_id(0)
    nsteps = pl.num_programs(0)
    t_col = t_ref[:, 0:1]
    th = t_col * tw1_ref[...] + tb1_ref[...]
    temb = jnp.dot(_gelu(th), tw2_ref[...],
                   preferred_element_type=jnp.float32) + tb2_ref[...]
    logits = (jnp.dot(x_ref[...], gw_ref[0], preferred_element_type=jnp.float32)
              + jnp.dot(p_ref[...], gw_ref[1], preferred_element_type=jnp.float32)
              + jnp.dot(temb, gw_ref[2], preferred_element_type=jnp.float32))
    lane = jax.lax.broadcasted_iota(jnp.int32, logits.shape, 1)
    lm = jnp.where(lane < E, logits, _NEG)
    m0 = jnp.max(lm, axis=1, keepdims=True)
    i0 = jnp.min(jnp.where(lm == m0, lane, 1 << 20), axis=1, keepdims=True)
    lm2 = jnp.where(lane == i0, _NEG, lm)
    m1 = jnp.max(lm2, axis=1, keepdims=True)
    i1 = jnp.min(jnp.where(lm2 == m1, lane, 1 << 20), axis=1, keepdims=True)
    b = jnp.exp(m1 - m0)
    denom = 1.0 + b
    s0 = 1.0 / denom
    s1 = b / denom
    swap = i1 < i0
    e_lo = jnp.where(swap, i1, i0)
    e_hi = jnp.where(swap, i0, i1)
    g_lo = jnp.where(swap, s1, s0)
    g_hi = jnp.where(swap, s0, s1)
    aux = (jnp.where(lane == 0, t_col, 0.0)
           + jnp.where(lane == 1, g_lo, 0.0)
           + jnp.where(lane == 2, g_hi, 0.0))
    aux_ref[...] = aux
    hot_lo = lane == e_lo
    hot_hi = lane == e_hi
    cnt = (jnp.sum(jnp.where(hot_lo & (g_lo > 0.0), 1.0, 0.0), axis=0, keepdims=True)
           + jnp.sum(jnp.where(hot_hi & (g_hi > 0.0), 1.0, 0.0), axis=0, keepdims=True))
    ld = (jnp.sum(jnp.where(hot_lo, g_lo, 0.0), axis=0, keepdims=True)
          + jnp.sum(jnp.where(hot_hi, g_hi, 0.0), axis=0, keepdims=True))
    upd = jnp.concatenate([cnt, ld], axis=0)

    @pl.when(step == 0)
    def _():
        stats_ref[...] = jnp.zeros_like(stats_ref)

    stats_ref[...] += upd

    @pl.when(step == nsteps - 1)
    def _():
        ldv = stats_ref[1:2, :]
        lane2 = jax.lax.broadcasted_iota(jnp.int32, ldv.shape, 1)
        msk = lane2 < E
        tot = jnp.sum(jnp.where(msk, ldv, 0.0))
        mean = tot / E
        var = jnp.sum(jnp.where(msk, (ldv - mean) ** 2, 0.0)) / (E - 1)
        loss_ref[...] = jnp.full((1, 1), 2.0 * var / (mean * mean + 1e-10),
                                 jnp.float32)


def _expert_body(D, S, HC, last_half,
                 jbv, ebv, firstv, activev, newwv, cume,
                 x_ref, aux_ref, tw2_ref, ep_ref, w1_ref, w2_ref, prev_ref,
                 out_ref, w1s_ref, w2s_ref, tws_ref):
    g = pl.program_id(0)

    @pl.when(newwv[g] == 1)
    def _():
        w1s_ref[...] = w1_ref[0].astype(jnp.bfloat16)
        w2s_ref[...] = w2_ref[0].astype(jnp.bfloat16)
        tws_ref[...] = tw2_ref[0].astype(jnp.bfloat16)

    @pl.when(firstv[g] == 1)
    def _():
        if last_half:
            out_ref[...] = prev_ref[...]
        else:
            out_ref[...] = jnp.zeros_like(out_ref)

    @pl.when(activev[g] == 1)
    def _():
        tw1 = ep_ref[0, 0:1, :]
        tb1 = ep_ref[0, 1:2, :]
        tb2 = ep_ref[0, 2:3, :]
        b2 = ep_ref[0, 3:4, :]
        b1 = ep_ref[0, 4:, :].reshape(1, 4 * D)[:, (HC if last_half else 0):][:, :HC]
        t_col = aux_ref[:, 0:1]
        th = t_col * tw1 + tb1
        temb = jnp.dot(_gelu(th).astype(jnp.bfloat16), tws_ref[...],
                       preferred_element_type=jnp.float32) + tb2
        e = ebv[g]
        lo = cume[e]
        hi = cume[e + 1]
        j = jbv[g]
        r0 = 2 * j * S + 2 * jax.lax.broadcasted_iota(jnp.int32, (S, 1), 0)
        r1 = r0 + 1
        gl = aux_ref[:, 1:2]
        gh = aux_ref[:, 2:3]
        w = (jnp.where((r0 >= lo) & (r0 < hi), gl, 0.0)
             + jnp.where((r1 >= lo) & (r1 < hi), gh, 0.0))
        xb = x_ref[...].astype(jnp.bfloat16)
        hh = _gelu(jnp.dot(xb, w1s_ref[:D, :], preferred_element_type=jnp.float32)
                   + jnp.dot(temb.astype(jnp.bfloat16), w1s_ref[D:, :],
                             preferred_element_type=jnp.float32)
                   + b1)
        oe = jnp.dot(hh.astype(jnp.bfloat16), w2s_ref[...],
                     preferred_element_type=jnp.float32)
        if last_half:
            oe = oe + b2
        out_ref[...] += w * oe


def kernel(x, prompt, t, te_w1, te_b1, te_w2, te_b2, gate_w,
           ex_te_w1, ex_te_b1, ex_te_w2, ex_te_b2,
           ex_m_w1, ex_m_b1, ex_m_w2, ex_m_b2):
    Bq, Nq, Dq = x.shape
    T = Bq * Nq
    D = Dq
    E = gate_w.shape[1]
    H = ex_m_w1.shape[2]

    SA = 256
    S = 512
    J = T // S
    G = J + E - 1

    xf = x.reshape(T, D)
    pf = prompt.reshape(T, D)
    t128 = jnp.broadcast_to(t.reshape(T, 1), (T, 128))
    gw3 = jnp.zeros((3, D, 128), jnp.float32).at[:, :, :E].set(
        gate_w.reshape(3, D, E))

    aux, stats, loss = pl.pallas_call(
        lambda *a: _gate_body(E, *a),
        grid=(T // SA,),
        in_specs=[
            pl.BlockSpec((SA, D), lambda i: (i, 0)),
            pl.BlockSpec((SA, D), lambda i: (i, 0)),
            pl.BlockSpec((SA, 128), lambda i: (i, 0)),
            pl.BlockSpec((1, D), lambda i: (0, 0)),
            pl.BlockSpec((1, D), lambda i: (0, 0)),
            pl.BlockSpec((D, D), lambda i: (0, 0)),
            pl.BlockSpec((1, D), lambda i: (0, 0)),
            pl.BlockSpec((3, D, 128), lambda i: (0, 0, 0)),
        ],
        out_specs=[
            pl.BlockSpec((SA, 128), lambda i: (i, 0)),
            pl.BlockSpec((2, 128), lambda i: (0, 0)),
            pl.BlockSpec((1, 1), lambda i: (0, 0)),
        ],
        out_shape=[
            jax.ShapeDtypeStruct((T, 128), jnp.float32),
            jax.ShapeDtypeStruct((2, 128), jnp.float32),
            jax.ShapeDtypeStruct((1, 1), jnp.float32),
        ],
    )(xf, pf, t128, te_w1, te_b1.reshape(1, D), te_w2, te_b2.reshape(1, D), gw3)

    counts = stats[0, :E].astype(jnp.int32)
    cum = jnp.cumsum(counts, dtype=jnp.int32)
    cume = jnp.concatenate([jnp.zeros((1,), jnp.int32), cum])
    rows_start = (2 * S) * jnp.arange(J, dtype=jnp.int32)
    ef = jnp.searchsorted(cum, rows_start, side='right').astype(jnp.int32)
    el = jnp.searchsorted(cum, rows_start + 2 * S - 1, side='right').astype(jnp.int32)
    ef_c = jnp.minimum(ef, E - 1)
    el_c = jnp.minimum(el, E - 1)
    npj = el_c - ef_c + 1
    off = jnp.concatenate(
        [jnp.zeros((1,), jnp.int32), jnp.cumsum(npj, dtype=jnp.int32)])
    total = off[-1]
    garr = jnp.arange(G, dtype=jnp.int32)
    jb = jnp.clip(jnp.searchsorted(off, garr, side='right').astype(jnp.int32) - 1,
                  0, J - 1)
    pin = garr - off[jb]
    eb = jnp.minimum(ef_c[jb] + pin, el_c[jb])
    active = ((garr < total) & (ef[jb] < E)).astype(jnp.int32)
    first = ((garr == off[jb]) & (garr < total)).astype(jnp.int32)

    neww = jnp.concatenate(
        [jnp.ones((1,), jnp.int32), (eb[1:] != eb[:-1]).astype(jnp.int32)])

    HC = H // 2
    epack = jnp.concatenate([
        ex_te_w1.reshape(E, 1, D),
        ex_te_b1.reshape(E, 1, D),
        ex_te_b2.reshape(E, 1, D),
        ex_m_b2.reshape(E, 1, D),
        ex_m_b1.reshape(E, 4, D),
    ], axis=1)

    def half_call(c, prev):
        grid_spec = pltpu.PrefetchScalarGridSpec(
            num_scalar_prefetch=6,
            grid=(G,),
            in_specs=[
                pl.BlockSpec((S, D), lambda g, jv, ev, fv, av, nv, cm: (jv[g], 0)),
                pl.BlockSpec((S, 128), lambda g, jv, ev, fv, av, nv, cm: (jv[g], 0)),
                pl.BlockSpec((1, D, D), lambda g, jv, ev, fv, av, nv, cm: (ev[g], 0, 0)),
                pl.BlockSpec((1, 8, D), lambda g, jv, ev, fv, av, nv, cm: (ev[g], 0, 0)),
                pl.BlockSpec((1, 2 * D, HC),
                             lambda g, jv, ev, fv, av, nv, cm: (ev[g], 0, c)),
                pl.BlockSpec((1, HC, D),
                             lambda g, jv, ev, fv, av, nv, cm: (ev[g], c, 0)),
                pl.BlockSpec((S, D), lambda g, jv, ev, fv, av, nv, cm: (jv[g], 0)),
            ],
            out_specs=pl.BlockSpec((S, D), lambda g, jv, ev, fv, av, nv, cm: (jv[g], 0)),
            scratch_shapes=[
                pltpu.VMEM((2 * D, HC), jnp.bfloat16),
                pltpu.VMEM((HC, D), jnp.bfloat16),
                pltpu.VMEM((D, D), jnp.bfloat16),
            ],
        )
        return pl.pallas_call(
            lambda *a: _expert_body(D, S, HC, c == 1, *a),
            grid_spec=grid_spec,
            out_shape=jax.ShapeDtypeStruct((T, D), jnp.float32),
        )(jb, eb, first, active, neww, cume,
          xf, aux, ex_te_w2, epack, ex_m_w1, ex_m_w2, prev)

    out = half_call(1, half_call(0, xf))

    output = out.reshape(Bq, Nq, Dq)
    moe_loss = loss[0, 0]
    return (output, moe_loss)


# speedup vs baseline: 1.4954x; 1.4501x over previous
import jax
import jax.numpy as jnp
from jax.experimental import pallas as pl
from jax.experimental.pallas import tpu as pltpu

_NEG = -1e30


def _gelu(x):
    return 0.5 * x * (1.0 + jax.lax.erf(x * 0.7071067811865476))


def _gate_body(E, x_ref, p_ref, t_ref, tw1_ref, tb1_ref, tw2_ref, tb2_ref,
               gw_ref, aux_ref, stats_ref, loss_ref):
    step = pl.program_id(0)
    nsteps = pl.num_programs(0)
    t_col = t_ref[:, 0:1]
    th = t_col * tw1_ref[...] + tb1_ref[...]
    temb = jnp.dot(_gelu(th), tw2_ref[...],
                   preferred_element_type=jnp.float32) + tb2_ref[...]
    logits = (jnp.dot(x_ref[...], gw_ref[0], preferred_element_type=jnp.float32)
              + jnp.dot(p_ref[...], gw_ref[1], preferred_element_type=jnp.float32)
              + jnp.dot(temb, gw_ref[2], preferred_element_type=jnp.float32))
    lane = jax.lax.broadcasted_iota(jnp.int32, logits.shape, 1)
    lm = jnp.where(lane < E, logits, _NEG)
    m0 = jnp.max(lm, axis=1, keepdims=True)
    i0 = jnp.min(jnp.where(lm == m0, lane, 1 << 20), axis=1, keepdims=True)
    lm2 = jnp.where(lane == i0, _NEG, lm)
    m1 = jnp.max(lm2, axis=1, keepdims=True)
    i1 = jnp.min(jnp.where(lm2 == m1, lane, 1 << 20), axis=1, keepdims=True)
    b = jnp.exp(m1 - m0)
    denom = 1.0 + b
    s0 = 1.0 / denom
    s1 = b / denom
    swap = i1 < i0
    e_lo = jnp.where(swap, i1, i0)
    e_hi = jnp.where(swap, i0, i1)
    g_lo = jnp.where(swap, s1, s0)
    g_hi = jnp.where(swap, s0, s1)
    aux = (jnp.where(lane == 0, t_col, 0.0)
           + jnp.where(lane == 1, g_lo, 0.0)
           + jnp.where(lane == 2, g_hi, 0.0))
    aux_ref[...] = aux
    hot_lo = lane == e_lo
    hot_hi = lane == e_hi
    cnt = (jnp.sum(jnp.where(hot_lo & (g_lo > 0.0), 1.0, 0.0), axis=0, keepdims=True)
           + jnp.sum(jnp.where(hot_hi & (g_hi > 0.0), 1.0, 0.0), axis=0, keepdims=True))
    ld = (jnp.sum(jnp.where(hot_lo, g_lo, 0.0), axis=0, keepdims=True)
          + jnp.sum(jnp.where(hot_hi, g_hi, 0.0), axis=0, keepdims=True))
    upd = jnp.concatenate([cnt, ld], axis=0)

    @pl.when(step == 0)
    def _():
        stats_ref[...] = jnp.zeros_like(stats_ref)

    stats_ref[...] += upd

    @pl.when(step == nsteps - 1)
    def _():
        ldv = stats_ref[1:2, :]
        lane2 = jax.lax.broadcasted_iota(jnp.int32, ldv.shape, 1)
        msk = lane2 < E
        tot = jnp.sum(jnp.where(msk, ldv, 0.0))
        mean = tot / E
        var = jnp.sum(jnp.where(msk, (ldv - mean) ** 2, 0.0)) / (E - 1)
        loss_ref[...] = jnp.full((1, 1), 2.0 * var / (mean * mean + 1e-10),
                                 jnp.float32)


def _expert_body(D, HC, CH, T, E, cume,
                 xh_ref, auxh_ref, tw2_ref, ep_ref, b1h_ref, w1_ref, w2_ref,
                 outh_ref, xs, auxs, outs, w1s, w2s, tws, sem):
    e = pl.program_id(0)
    c = pl.program_id(1)
    step = e * 2 + c

    @pl.when(step == 0)
    def _():
        cp = pltpu.make_async_copy(xh_ref, xs, sem)
        cp.start()
        cp.wait()
        cp2 = pltpu.make_async_copy(auxh_ref, auxs, sem)
        cp2.start()
        cp2.wait()
        outs[...] = jnp.zeros_like(outs)

    w1s[...] = w1_ref[0].astype(jnp.bfloat16)
    w2s[...] = w2_ref[0].astype(jnp.bfloat16)
    tws[...] = tw2_ref[0].astype(jnp.bfloat16)

    lo = cume[e]
    hi = cume[e + 1]
    lo2 = jax.lax.div(lo, 2)
    hi2t = jax.lax.div(hi + 1, 2)
    astart = jax.lax.div(lo2, 8) * 8
    n = jnp.maximum(hi2t - astart, 0)
    nch = jax.lax.div(n + CH - 1, CH)
    cf = c.astype(jnp.float32)

    tw1 = ep_ref[0, 0:1, :]
    tb1 = ep_ref[0, 1:2, :]
    tb2 = ep_ref[0, 2:3, :]
    b2 = ep_ref[0, 3:4, :]
    b1 = b1h_ref[0, 0, 0:1, :]

    def chunk(k, carry):
        s_real = astart + k * CH
        s = jnp.minimum(s_real, T - CH)
        tok = s + jax.lax.broadcasted_iota(jnp.int32, (CH, 1), 0)
        auxc = auxs[pl.ds(s, CH), :]
        t_col = auxc[:, 0:1]
        gl = auxc[:, 1:2]
        gh = auxc[:, 2:3]
        temb = jnp.dot(_gelu(t_col * tw1 + tb1).astype(jnp.bfloat16), tws[...],
                       preferred_element_type=jnp.float32) + tb2
        r0 = 2 * tok
        r1 = r0 + 1
        w = (jnp.where((r0 >= lo) & (r0 < hi), gl, 0.0)
             + jnp.where((r1 >= lo) & (r1 < hi), gh, 0.0))
        w = jnp.where(tok >= s_real, w, 0.0)
        xc = xs[pl.ds(s, CH), :].astype(jnp.bfloat16)
        hh = _gelu(jnp.dot(xc, w1s[:D, :], preferred_element_type=jnp.float32)
                   + jnp.dot(temb.astype(jnp.bfloat16), w1s[D:, :],
                             preferred_element_type=jnp.float32)
                   + b1)
        oe = jnp.dot(hh.astype(jnp.bfloat16), w2s[...],
                     preferred_element_type=jnp.float32) + b2 * cf
        outs[pl.ds(s, CH), :] += w * oe
        return carry

    jax.lax.fori_loop(0, nch, chunk, 0)

    @pl.when(step == 2 * E - 1)
    def _():
        cp = pltpu.make_async_copy(outs, outh_ref, sem)
        cp.start()
        cp.wait()


def kernel(x, prompt, t, te_w1, te_b1, te_w2, te_b2, gate_w,
           ex_te_w1, ex_te_b1, ex_te_w2, ex_te_b2,
           ex_m_w1, ex_m_b1, ex_m_w2, ex_m_b2):
    Bq, Nq, Dq = x.shape
    T = Bq * Nq
    D = Dq
    E = gate_w.shape[1]
    H = ex_m_w1.shape[2]

    SA = 256
    CH = 128
    HC = H // 2

    xf = x.reshape(T, D)
    pf = prompt.reshape(T, D)
    t128 = jnp.broadcast_to(t.reshape(T, 1), (T, 128))
    gw3 = jnp.zeros((3, D, 128), jnp.float32).at[:, :, :E].set(
        gate_w.reshape(3, D, E))

    aux, stats, loss = pl.pallas_call(
        lambda *a: _gate_body(E, *a),
        grid=(T // SA,),
        in_specs=[
            pl.BlockSpec((SA, D), lambda i: (i, 0)),
            pl.BlockSpec((SA, D), lambda i: (i, 0)),
            pl.BlockSpec((SA, 128), lambda i: (i, 0)),
            pl.BlockSpec((1, D), lambda i: (0, 0)),
            pl.BlockSpec((1, D), lambda i: (0, 0)),
            pl.BlockSpec((D, D), lambda i: (0, 0)),
            pl.BlockSpec((1, D), lambda i: (0, 0)),
            pl.BlockSpec((3, D, 128), lambda i: (0, 0, 0)),
        ],
        out_specs=[
            pl.BlockSpec((SA, 128), lambda i: (i, 0)),
            pl.BlockSpec((2, 128), lambda i: (0, 0)),
            pl.BlockSpec((1, 1), lambda i: (0, 0)),
        ],
        out_shape=[
            jax.ShapeDtypeStruct((T, 128), jnp.float32),
            jax.ShapeDtypeStruct((2, 128), jnp.float32),
            jax.ShapeDtypeStruct((1, 1), jnp.float32),
        ],
    )(xf, pf, t128, te_w1, te_b1.reshape(1, D), te_w2, te_b2.reshape(1, D), gw3)

    counts = stats[0, :E].astype(jnp.int32)
    cume = jnp.concatenate(
        [jnp.zeros((1,), jnp.int32), jnp.cumsum(counts, dtype=jnp.int32)])

    epack = jnp.concatenate([
        ex_te_w1.reshape(E, 1, D),
        ex_te_b1.reshape(E, 1, D),
        ex_te_b2.reshape(E, 1, D),
        ex_m_b2.reshape(E, 1, D),
    ], axis=1)
    b1h = ex_m_b1.reshape(E, 2, 1, HC)

    grid_spec = pltpu.PrefetchScalarGridSpec(
        num_scalar_prefetch=1,
        grid=(E, 2),
        in_specs=[
            pl.BlockSpec(memory_space=pl.ANY),
            pl.BlockSpec(memory_space=pl.ANY),
            pl.BlockSpec((1, D, D), lambda e, c, cm: (e, 0, 0)),
            pl.BlockSpec((1, 4, D), lambda e, c, cm: (e, 0, 0)),
            pl.BlockSpec((1, 1, 1, HC), lambda e, c, cm: (e, c, 0, 0)),
            pl.BlockSpec((1, 2 * D, HC), lambda e, c, cm: (e, 0, c)),
            pl.BlockSpec((1, HC, D), lambda e, c, cm: (e, c, 0)),
        ],
        out_specs=pl.BlockSpec(memory_space=pl.ANY),
        scratch_shapes=[
            pltpu.VMEM((T, D), jnp.float32),
            pltpu.VMEM((T, 128), jnp.float32),
            pltpu.VMEM((T, D), jnp.float32),
            pltpu.VMEM((2 * D, HC), jnp.bfloat16),
            pltpu.VMEM((HC, D), jnp.bfloat16),
            pltpu.VMEM((D, D), jnp.bfloat16),
            pltpu.SemaphoreType.DMA,
        ],
    )
    out = pl.pallas_call(
        lambda *a: _expert_body(D, HC, CH, T, E, *a),
        grid_spec=grid_spec,
        out_shape=jax.ShapeDtypeStruct((T, D), jnp.float32),
    )(cume, xf, aux, ex_te_w2, epack, b1h, ex_m_w1, ex_m_w2)

    output = out.reshape(Bq, Nq, Dq)
    moe_loss = loss[0, 0]
    return (output, moe_loss)
